# baseline (speedup 1.0000x reference)
import jax
import jax.numpy as jnp
from jax import lax
from jax.experimental import pallas as pl
from jax.experimental.pallas import tpu as pltpu

_NUM_SLOTS = 8192
_NUM_UPD = 4096
_D = 128


def _body(src_ref, a0_ref, a3_ref, out_ref):
    i = pl.program_id(0)
    s = src_ref[i]
    row = jnp.where(s >= 0, a3_ref[...], a0_ref[...])
    r = lax.broadcasted_iota(jnp.int32, (1, _D, _D), 1)
    c = lax.broadcasted_iota(jnp.int32, (1, _D, _D), 2)
    out_ref[...] = jnp.where(r == c, 0.0, row)


@jax.jit
def kernel(arg0_1, arg1_1, arg2_1, arg3_1):
    del arg2_1
    idx = arg1_1.astype(jnp.int32)
    src = jnp.full((_NUM_SLOTS,), -1, jnp.int32).at[idx].max(
        jnp.arange(_NUM_UPD, dtype=jnp.int32)
    )

    grid_spec = pltpu.PrefetchScalarGridSpec(
        num_scalar_prefetch=1,
        grid=(_NUM_SLOTS,),
        in_specs=[
            pl.BlockSpec((1, _D, _D), lambda i, src: (i, 0, 0)),
            pl.BlockSpec(
                (1, _D, _D), lambda i, src: (jnp.maximum(src[i], 0), 0, 0)
            ),
        ],
        out_specs=pl.BlockSpec((1, _D, _D), lambda i, src: (i, 0, 0)),
    )
    return pl.pallas_call(
        _body,
        grid_spec=grid_spec,
        out_shape=jax.ShapeDtypeStruct((_NUM_SLOTS, _D, _D), jnp.float32),
        compiler_params=pltpu.CompilerParams(
            dimension_semantics=("arbitrary",),
        ),
    )(src, arg0_1, arg3_1)


# speedup vs baseline: 5.5194x; 5.5194x over previous
import functools

import jax
import jax.numpy as jnp
from jax import lax
from jax.experimental import pallas as pl
from jax.experimental.pallas import tpu as pltpu
from jax.experimental.pallas import tpu_sc as plsc

_NUM_SLOTS = 8192
_NUM_UPD = 4096
_D = 128

_NC = 2
_NS = 16
_NW = _NC * _NS

_K = 4
_L = 16
_PAD = _L


_TC_BLOCK = 16


def _tc_body(a0_ref, out_ref):
    r = lax.broadcasted_iota(jnp.int32, (_TC_BLOCK, _D, _D), 1)
    c = lax.broadcasted_iota(jnp.int32, (_TC_BLOCK, _D, _D), 2)
    out_ref[...] = jnp.where(r == c, 0.0, a0_ref[...])


def _tc_diagzero(arg0):
    return pl.pallas_call(
        _tc_body,
        grid=(_NUM_SLOTS // _TC_BLOCK,),
        in_specs=[pl.BlockSpec((_TC_BLOCK, _D, _D), lambda i: (i, 0, 0))],
        out_specs=pl.BlockSpec((_TC_BLOCK, _D, _D), lambda i: (i, 0, 0)),
        out_shape=jax.ShapeDtypeStruct((_NUM_SLOTS, _D, _D), jnp.float32),
        compiler_params=pltpu.CompilerParams(
            dimension_semantics=("arbitrary",),
        ),
    )(arg0)


def _sload(ref, i):
    return ref[pl.ds(i, _L)][0]


def _sstore(ref, i, val, lane0):
    plsc.store_scatter(
        ref,
        [jnp.full((_L,), i, jnp.int32)],
        jnp.full((_L,), val, ref.dtype),
        mask=lane0,
    )


_sc_mesh = plsc.VectorSubcoreMesh(
    core_axis_name="c", subcore_axis_name="s", num_cores=_NC, num_subcores=_NS
)


@functools.partial(
    pl.kernel,
    out_type=(),
    mesh=_sc_mesh,
    compiler_params=pltpu.CompilerParams(needs_layout_passes=False),
    scratch_types=[
        pltpu.VMEM((_NUM_UPD + _PAD,), jnp.int32),
        pltpu.VMEM((_NUM_UPD + _PAD,), jnp.int32),
        pltpu.VMEM((_NUM_UPD + _PAD,), jnp.int32),
        pltpu.VMEM((_NUM_UPD + _PAD,), jnp.int32),
        pltpu.VMEM((_NUM_UPD + _PAD,), jnp.int32),
        pltpu.VMEM((_NUM_SLOTS + _PAD,), jnp.int32),
        pltpu.VMEM((_K, _D, _D), jnp.float32),
        pltpu.SemaphoreType.DMA,
        pltpu.SemaphoreType.DMA,
    ],
)
def _sc_scatter(idx_hbm, arg3_hbm, out_hbm,
                idx_v, myj, myd, wqj, wqd, last, bufs, in_sem, out_sem):
    wid = lax.axis_index("s") * _NC + lax.axis_index("c")
    ii0 = lax.iota(jnp.int32, _L)
    lane0 = ii0 == 0

    pltpu.sync_copy(idx_hbm, idx_v.at[pl.ds(0, _NUM_UPD)])

    def sel_body(j, cnt):
        d = _sload(idx_v, j)
        mine = (d & (_NW - 1)) == wid

        @pl.when(mine)
        def _():
            _sstore(myd, cnt, d, lane0)
            _sstore(myj, cnt, j, lane0)

        return cnt + mine.astype(jnp.int32)

    cnt = lax.fori_loop(0, _NUM_UPD, sel_body, jnp.int32(0))

    def last_body(p, _):
        _sstore(last, _sload(myd, p), p, lane0)
        return 0

    lax.fori_loop(0, cnt, last_body, 0)

    def win_body(p, w):
        d = _sload(myd, p)
        is_w = _sload(last, d) == p

        @pl.when(is_w)
        def _():
            _sstore(wqj, w, _sload(myj, p), lane0)
            _sstore(wqd, w, d, lane0)

        return w + is_w.astype(jnp.int32)

    wcnt = lax.fori_loop(0, cnt, win_body, jnp.int32(0))

    nchunks = (wcnt + _K - 1) // _K

    def pad_body(p, _):
        _sstore(wqj, p, _sload(wqj, wcnt - 1), lane0)
        _sstore(wqd, p, _sload(wqd, wcnt - 1), lane0)
        return 0

    lax.fori_loop(wcnt, nchunks * _K, pad_body, 0)

    zero16 = jnp.zeros((_L,), jnp.float32)

    def chunk_body(ch, _):
        base = ch * _K
        ins = [
            pltpu.async_copy(
                arg3_hbm.at[_sload(wqj, base + t)], bufs.at[t], in_sem
            )
            for t in range(_K)
        ]
        for h in ins:
            h.wait()
        for t in range(_K):
            for b in range(_D // _L):
                ii = ii0 + (_L * b)
                plsc.store_scatter(bufs.at[t], [ii, ii], zero16)
        outs = [
            pltpu.async_copy(
                bufs.at[t], out_hbm.at[_sload(wqd, base + t)], out_sem
            )
            for t in range(_K)
        ]
        for h in outs:
            h.wait()
        return 0

    lax.fori_loop(0, nchunks, chunk_body, 0)


@jax.jit
def kernel(arg0_1, arg1_1, arg2_1, arg3_1):
    del arg2_1
    idx = arg1_1.astype(jnp.int32)
    outz = _tc_diagzero(arg0_1)
    out_ref = jax.new_ref(outz)
    _sc_scatter(idx, arg3_1, out_ref)
    return out_ref[...]


# speedup vs baseline: 5.6354x; 1.0210x over previous
import functools

import jax
import jax.numpy as jnp
from jax import lax
from jax.experimental import pallas as pl
from jax.experimental.pallas import tpu as pltpu
from jax.experimental.pallas import tpu_sc as plsc

_NUM_SLOTS = 8192
_NUM_UPD = 4096
_D = 128

_NC = 2
_NS = 16
_NW = _NC * _NS

_K = 3
_L = 16
_PAD = 4 * _K + _L


_TC_BLOCK = 16


def _tc_body(a0_ref, out_ref):
    r = lax.broadcasted_iota(jnp.int32, (_TC_BLOCK, _D, _D), 1)
    c = lax.broadcasted_iota(jnp.int32, (_TC_BLOCK, _D, _D), 2)
    out_ref[...] = jnp.where(r == c, 0.0, a0_ref[...])


def _tc_diagzero(arg0):
    return pl.pallas_call(
        _tc_body,
        grid=(_NUM_SLOTS // _TC_BLOCK,),
        in_specs=[pl.BlockSpec((_TC_BLOCK, _D, _D), lambda i: (i, 0, 0))],
        out_specs=pl.BlockSpec((_TC_BLOCK, _D, _D), lambda i: (i, 0, 0)),
        out_shape=jax.ShapeDtypeStruct((_NUM_SLOTS, _D, _D), jnp.float32),
        compiler_params=pltpu.CompilerParams(
            dimension_semantics=("arbitrary",),
        ),
    )(arg0)


def _sload(ref, i):
    return ref[pl.ds(i, _L)][0]


def _sstore(ref, i, val, lane0):
    plsc.store_scatter(
        ref,
        [jnp.full((_L,), i, jnp.int32)],
        jnp.full((_L,), val, ref.dtype),
        mask=lane0,
    )


_sc_mesh = plsc.VectorSubcoreMesh(
    core_axis_name="c", subcore_axis_name="s", num_cores=_NC, num_subcores=_NS
)


@functools.partial(
    pl.kernel,
    out_type=(),
    mesh=_sc_mesh,
    compiler_params=pltpu.CompilerParams(needs_layout_passes=False),
    scratch_types=[
        pltpu.VMEM((_NUM_UPD + _PAD,), jnp.int32),
        pltpu.VMEM((_NUM_UPD + _PAD,), jnp.int32),
        pltpu.VMEM((_NUM_UPD + _PAD,), jnp.int32),
        pltpu.VMEM((_NUM_SLOTS + _L,), jnp.int32),
        pltpu.VMEM((2 * _K, _D, _D), jnp.float32),
        pltpu.SemaphoreType.DMA,
        pltpu.SemaphoreType.DMA,
        pltpu.SemaphoreType.DMA,
        pltpu.SemaphoreType.DMA,
    ],
)
def _sc_scatter(idx_hbm, arg3_hbm, out_hbm,
                idx_v, myj, myd, last, bufs,
                in_s0, in_s1, out_s0, out_s1):
    wid = lax.axis_index("s") * _NC + lax.axis_index("c")
    ii0 = lax.iota(jnp.int32, _L)
    lane0 = ii0 == 0
    in_sems = [in_s0, in_s1]
    out_sems = [out_s0, out_s1]

    pltpu.sync_copy(idx_hbm, idx_v.at[pl.ds(0, _NUM_UPD)])

    def sel_body(j, cnt):
        d = _sload(idx_v, j)
        mine = (d & (_NW - 1)) == wid

        @pl.when(mine)
        def _():
            _sstore(myd, cnt, d, lane0)
            _sstore(myj, cnt, j, lane0)

        return cnt + mine.astype(jnp.int32)

    cnt = lax.fori_loop(0, _NUM_UPD, sel_body, jnp.int32(0))

    def last_body(p, _):
        _sstore(last, _sload(myd, p), p, lane0)
        return 0

    lax.fori_loop(0, cnt, last_body, 0)

    def win_body(p, w):
        d = _sload(myd, p)
        j = _sload(myj, p)
        is_w = _sload(last, d) == p

        @pl.when(is_w)
        def _():
            _sstore(myj, w, j, lane0)
            _sstore(myd, w, d, lane0)

        return w + is_w.astype(jnp.int32)

    wcnt = lax.fori_loop(0, cnt, win_body, jnp.int32(0))

    nchunks = (wcnt + _K - 1) // _K
    npairs = (nchunks + 1) // 2

    def pad_body(p, _):
        _sstore(myj, p, _sload(myj, wcnt - 1), lane0)
        _sstore(myd, p, _sload(myd, wcnt - 1), lane0)
        return 0

    lax.fori_loop(wcnt, (2 * npairs + 1) * _K, pad_body, 0)

    zero16 = jnp.zeros((_L,), jnp.float32)

    def fire_in(c, g):
        for t in range(_K):
            pltpu.async_copy(
                arg3_hbm.at[_sload(myj, c * _K + t)],
                bufs.at[g * _K + t],
                in_sems[g],
            )

    def drain_in(g):
        for t in range(_K):
            pltpu.make_async_copy(
                arg3_hbm.at[0], bufs.at[g * _K + t], in_sems[g]
            ).wait()

    def fire_out(c, g):
        for t in range(_K):
            pltpu.async_copy(
                bufs.at[g * _K + t],
                out_hbm.at[_sload(myd, c * _K + t)],
                out_sems[g],
            )

    def drain_out(g):
        for t in range(_K):
            pltpu.make_async_copy(
                bufs.at[g * _K + t], out_hbm.at[0], out_sems[g]
            ).wait()

    def zero_diag(g):
        for t in range(_K):
            for b in range(_D // _L):
                ii = ii0 + (_L * b)
                plsc.store_scatter(bufs.at[g * _K + t], [ii, ii], zero16)

    @pl.when(wcnt > 0)
    def _pipeline():
        fire_in(0, 0)

        def pair_body(cp, _):
            c0 = 2 * cp
            drain_in(0)
            zero_diag(0)
            fire_out(c0, 0)

            @pl.when(cp >= 1)
            def _():
                drain_out(1)

            fire_in(c0 + 1, 1)
            drain_in(1)
            zero_diag(1)
            fire_out(c0 + 1, 1)
            drain_out(0)
            fire_in(c0 + 2, 0)
            return 0

        lax.fori_loop(0, npairs, pair_body, 0)

        drain_out(1)
        drain_in(0)


@jax.jit
def kernel(arg0_1, arg1_1, arg2_1, arg3_1):
    del arg2_1
    idx = arg1_1.astype(jnp.int32)
    outz = _tc_diagzero(arg0_1)
    out_ref = jax.new_ref(outz)
    _sc_scatter(idx, arg3_1, out_ref)
    return out_ref[...]


# speedup vs baseline: 9.1123x; 1.6170x over previous
import functools

import jax
import jax.numpy as jnp
from jax import lax
from jax.experimental import pallas as pl
from jax.experimental.pallas import tpu as pltpu
from jax.experimental.pallas import tpu_sc as plsc

_NUM_SLOTS = 8192
_NUM_UPD = 4096
_D = 128

_NC = 2
_NS = 16
_NW = _NC * _NS

_SLOTS_W = _NUM_SLOTS // _NW

_K = 3
_L = 16
_NCHUNKS = (_SLOTS_W + _K - 1) // _K
_NPAIRS = (_NCHUNKS + 1) // 2
_PW = 2 * _NPAIRS * _K
_PAD = _L + _K * 4


def _sload(ref, i):
    return ref[pl.ds(i, _L)][0]


def _sstore(ref, i, val, lane0):
    plsc.store_scatter(
        ref,
        [jnp.full((_L,), i, jnp.int32)],
        jnp.full((_L,), val, ref.dtype),
        mask=lane0,
    )


_sc_mesh = plsc.VectorSubcoreMesh(
    core_axis_name="c", subcore_axis_name="s", num_cores=_NC, num_subcores=_NS
)


@functools.partial(
    pl.kernel,
    out_type=jax.ShapeDtypeStruct((_NUM_SLOTS, _D, _D), jnp.float32),
    mesh=_sc_mesh,
    compiler_params=pltpu.CompilerParams(needs_layout_passes=False),
    scratch_types=[
        pltpu.VMEM((_NUM_UPD + _L,), jnp.int32),
        pltpu.VMEM((_PW + _PAD,), jnp.int32),
        pltpu.VMEM((_PW + _PAD,), jnp.int32),
        pltpu.VMEM((2 * _K, _D, _D), jnp.float32),
        pltpu.SemaphoreType.DMA,
        pltpu.SemaphoreType.DMA,
        pltpu.SemaphoreType.DMA,
        pltpu.SemaphoreType.DMA,
    ],
)
def _sc_update(idx_hbm, arg0_hbm, arg3_hbm, out_hbm,
               idx_v, jwin, dst, bufs, in_s0, in_s1, out_s0, out_s1):
    wid = lax.axis_index("s") * _NC + lax.axis_index("c")
    ii0 = lax.iota(jnp.int32, _L)
    lane0 = ii0 == 0
    in_sems = [in_s0, in_s1]
    out_sems = [out_s0, out_s1]

    pltpu.sync_copy(idx_hbm, idx_v.at[pl.ds(0, _NUM_UPD)])

    for b in range((_PW + _PAD) // _L):
        sl = pl.ds(b * _L, _L)
        jwin[sl] = jnp.full((_L,), -1, jnp.int32)
        pos = jnp.minimum(ii0 + (b * _L), _SLOTS_W - 1)
        dst[sl] = pos * _NW + wid

    def sel_body(j, _):
        d = _sload(idx_v, j)
        mine = (d & (_NW - 1)) == wid

        @pl.when(mine)
        def _():
            _sstore(jwin, lax.shift_right_logical(d, 5), j, lane0)

        return 0

    lax.fori_loop(0, _NUM_UPD, sel_body, 0)

    jlast = _sload(jwin, _SLOTS_W - 1)

    def pad_body(p, _):
        _sstore(jwin, p, jlast, lane0)
        return 0

    lax.fori_loop(_SLOTS_W, _PW, pad_body, 0)

    zero16 = jnp.zeros((_L,), jnp.float32)

    def fire_in(c, g):
        for t in range(_K):
            p = c * _K + t
            j = _sload(jwin, p)
            d = _sload(dst, p)

            @pl.when(j >= 0)
            def _():
                pltpu.async_copy(
                    arg3_hbm.at[j], bufs.at[g * _K + t], in_sems[g]
                )

            @pl.when(j < 0)
            def _():
                pltpu.async_copy(
                    arg0_hbm.at[d], bufs.at[g * _K + t], in_sems[g]
                )

    def drain_in(g):
        for t in range(_K):
            pltpu.make_async_copy(
                arg0_hbm.at[0], bufs.at[g * _K + t], in_sems[g]
            ).wait()

    def fire_out(c, g):
        for t in range(_K):
            pltpu.async_copy(
                bufs.at[g * _K + t],
                out_hbm.at[_sload(dst, c * _K + t)],
                out_sems[g],
            )

    def drain_out(g):
        for t in range(_K):
            pltpu.make_async_copy(
                bufs.at[g * _K + t], out_hbm.at[0], out_sems[g]
            ).wait()

    def zero_diag(g):
        for t in range(_K):
            for b in range(_D // _L):
                ii = ii0 + (_L * b)
                plsc.store_scatter(bufs.at[g * _K + t], [ii, ii], zero16)

    fire_in(0, 0)

    def pair_body(cp, _):
        c0 = 2 * cp
        drain_in(0)
        zero_diag(0)
        fire_out(c0, 0)

        @pl.when(cp >= 1)
        def _():
            drain_out(1)

        fire_in(c0 + 1, 1)
        drain_in(1)
        zero_diag(1)
        fire_out(c0 + 1, 1)
        drain_out(0)
        fire_in(c0 + 2, 0)
        return 0

    lax.fori_loop(0, _NPAIRS, pair_body, 0)

    drain_out(1)
    drain_in(0)


@jax.jit
def kernel(arg0_1, arg1_1, arg2_1, arg3_1):
    del arg2_1
    idx = arg1_1.astype(jnp.int32)
    return _sc_update(idx, arg0_1, arg3_1)
